# Initial kernel scaffold; baseline (speedup 1.0000x reference)
import jax
import jax.numpy as jnp
from jax.experimental import pallas as pl


def kernel(atom_types, bond_fea, nbr_list, target_index, emb, Wc, bc, Wf, bf, ga, ba, gb, bb, Wd, bd):
    raise NotImplementedError("write your pallas kernel here")


# speedup vs baseline: 7.9317x; 7.9317x over previous
import functools

import jax
import jax.numpy as jnp
from jax import lax
from jax.experimental import pallas as pl
from jax.experimental.pallas import tpu as pltpu
from jax.experimental.pallas import tpu_sc as plsc

B, N, M, F, BF, NC, N0 = 2, 10000, 16, 128, 16, 3, 1000
EPS = 1e-3

_NUM_CORES = 2
_NUM_SUBCORES = 16
_NW = _NUM_CORES * _NUM_SUBCORES


@functools.lru_cache(None)
def _sc_gather(T, R, C):
    bpw = R // _NW
    assert R % _NW == 0 and bpw % C == 0 and C % 8 == 0 and C <= 128
    nchunks = bpw // C
    mesh = plsc.VectorSubcoreMesh(core_axis_name="c", subcore_axis_name="s")

    def body(table_hbm, idx_hbm, out_hbm, idx_v, rows_v, sem):
        wid = lax.axis_index("s") * _NUM_CORES + lax.axis_index("c")
        base = wid * bpw

        def chunk(c, carry):
            r0 = base + c * C
            pltpu.sync_copy(idx_hbm.at[pl.ds(r0, C)], idx_v)
            pltpu.async_copy(table_hbm.at[idx_v], rows_v, sem).wait()
            pltpu.sync_copy(rows_v, out_hbm.at[pl.ds(r0, C)])
            return carry

        lax.fori_loop(0, nchunks, chunk, 0)

    return pl.kernel(
        body,
        mesh=mesh,
        out_type=jax.ShapeDtypeStruct((R, F), jnp.float32),
        scratch_types=[
            pltpu.VMEM((C,), jnp.int32),
            pltpu.VMEM((C, F), jnp.float32),
            pltpu.SemaphoreType.DMA,
        ],
    )


_NB = 400


def _combine_body(g_ref, bond_ref, x_ref, As_ref, b1_ref, An_ref, Ab_ref,
                  wfn_ref, wfb_ref, c2_ref, b2_ref, o_ref):
    x_blk = x_ref[...]
    g2 = g_ref[...]
    bond2 = bond_ref[...]
    xn = jnp.dot(g2, An_ref[...], preferred_element_type=jnp.float32)
    bcr = jnp.dot(bond2, Ab_ref[...], preferred_element_type=jnp.float32)
    xs = jnp.dot(x_blk, As_ref[...], preferred_element_type=jnp.float32)
    xs = xs + b1_ref[...]
    pre = (xn + bcr).reshape(_NB, M, F) + xs[:, None, :]
    core = jnp.maximum(pre, 0.0)
    fn = jnp.sum(g2.reshape(_NB, M, F) * wfn_ref[...][None, :, :], axis=-1)
    fb = jnp.sum(bond2.reshape(_NB, M, BF) * wfb_ref[...][None, :, :], axis=-1)
    filt = fn + fb
    mx = jnp.max(filt, axis=1, keepdims=True)
    e = jnp.exp(filt - mx)
    w = e / jnp.sum(e, axis=1, keepdims=True)
    sacc = jnp.sum(w[:, :, None] * core, axis=1)
    o_ref[...] = jnp.maximum(x_blk + c2_ref[...] * sacc + b2_ref[...], 0.0)


@functools.lru_cache(None)
def _combine_call():
    R = B * N
    grid = (R // _NB,)
    full = lambda i: (0, 0)
    return pl.pallas_call(
        _combine_body,
        grid=grid,
        in_specs=[
            pl.BlockSpec((_NB * M, F), lambda i: (i, 0)),
            pl.BlockSpec((_NB * M, BF), lambda i: (i, 0)),
            pl.BlockSpec((_NB, F), lambda i: (i, 0)),
            pl.BlockSpec((F, F), full),
            pl.BlockSpec((1, F), full),
            pl.BlockSpec((F, F), full),
            pl.BlockSpec((BF, F), full),
            pl.BlockSpec((1, F), full),
            pl.BlockSpec((1, BF), full),
            pl.BlockSpec((1, F), full),
            pl.BlockSpec((1, F), full),
        ],
        out_specs=pl.BlockSpec((_NB, F), lambda i: (i, 0)),
        out_shape=jax.ShapeDtypeStruct((R, F), jnp.float32),
    )


def _head_body(c_ref, wd_ref, bd_ref, o_ref):
    crys = jnp.maximum(c_ref[...], 0.0)
    o = jnp.dot(crys, wd_ref[...], preferred_element_type=jnp.float32)
    o_ref[...] = jnp.maximum(o + bd_ref[...], 0.0)


@functools.lru_cache(None)
def _head_call(R):
    return pl.pallas_call(
        _head_body,
        out_shape=jax.ShapeDtypeStruct((R, F), jnp.float32),
    )


def _pad_to(v, r):
    return jnp.pad(v, (0, r - v.shape[0]))


def kernel(atom_types, bond_fea, nbr_list, target_index, emb, Wc, bc, Wf,
           bf, ga, ba, gb, bb, Wd, bd):
    inv = 1.0 / jnp.sqrt(1.0 + EPS)
    ga_s = ga * inv
    A_self = Wc[:, :F, :] * ga_s[:, None, :]
    A_nbr = Wc[:, F:2 * F, :] * ga_s[:, None, :]
    A_bond = Wc[:, 2 * F:, :] * ga_s[:, None, :]
    bias1 = ga_s * bc + ba
    wfn = Wf[:, F:2 * F, 0]
    wfb = Wf[:, 2 * F:, 0]
    c2 = gb * (inv / M)
    b2 = bb

    RA = 20480
    at_flat = _pad_to(atom_types.astype(jnp.int32).reshape(-1), RA)
    x = _sc_gather(100, RA, 128)(emb, at_flat)[:B * N]

    offs = jnp.arange(B, dtype=jnp.int32) * N
    RE = 323584
    nbr_flat = _pad_to(
        (nbr_list.astype(jnp.int32) + offs[:, None, None]).reshape(-1), RE)
    bond2 = bond_fea.reshape(B * N * M, BF)

    combine = _combine_call()
    for i in range(NC):
        g = _sc_gather(B * N, RE, 128)(x, nbr_flat)[:B * N * M]
        x = combine(g, bond2, x, A_self[i], bias1[i][None], A_nbr[i],
                    A_bond[i], wfn[i][None], wfb[i][None], c2[i][None],
                    b2[i][None])

    RT = 2048
    tgt_flat = _pad_to(
        (target_index.astype(jnp.int32) + offs[:, None]).reshape(-1), RT)
    crys = _sc_gather(B * N, RT, 64)(x, tgt_flat)
    out = _head_call(RT)(crys, Wd, bd[None])
    return out[:B * N0].reshape(B, N0, F)
